# baseline (speedup 1.0000x reference)
import functools

import jax
import jax.numpy as jnp
from jax import lax
from jax.experimental import pallas as pl
from jax.experimental.pallas import tpu as pltpu
from jax.experimental.pallas import tpu_sc as plsc

H = 12
M_LANDMARKS = 256
TOPK = 4
NEG = -1e30
BIG = 1024.0
BF = jnp.bfloat16


def _mm(a, b, dims):
    return jax.lax.dot_general(a.astype(BF), b.astype(BF), (dims, ((), ())),
                               preferred_element_type=jnp.float32)


def _qkv_kernel(x_ref, w_ref, o_ref):
    o_ref[...] = _mm(x_ref[...], w_ref[...], ((1,), (1,)))


def _rs_kernel(q_ref, k_ref, o_ref, *, seg, hd):
    n = k_ref.shape[0]
    m = n // seg
    for half in range(2):
        sl = slice(half * hd, (half + 1) * hd)
        k = k_ref[:, sl]
        cent = jnp.mean(k.reshape(m, seg, hd), axis=1)
        rs = _mm(q_ref[:, sl], cent, ((1,), (1,)))
        o_ref[half] = rs.T


def _sc_topk(rs_ref, idx_ref, slab, buf, *, groups_per_w, gph, m):
    wid = lax.axis_index("s") * 2 + lax.axis_index("c")

    def group_body(t, carry):
        gg = wid * groups_per_w + t
        h = gg // gph
        g = gg - h * gph
        pltpu.sync_copy(rs_ref.at[h, :, pl.ds(g * 128, 128)], slab)

        neg = jnp.full((16,), NEG, jnp.float32)
        iinit = jnp.full((16,), m, jnp.int32)

        for sg in range(8):
            def seg_body(j, c, sg=sg):
                t0, t1, t2, t3, i0, i1, i2, i3 = c
                v = slab[j, pl.ds(sg * 16, 16)]
                jv = jnp.zeros((16,), jnp.int32) + j
                g0 = v > t0
                g1 = v > t1
                g2 = v > t2
                g3 = v > t3
                nt0 = jnp.where(g0, v, t0)
                ni0 = jnp.where(g0, jv, i0)
                nt1 = jnp.where(g0, t0, jnp.where(g1, v, t1))
                ni1 = jnp.where(g0, i0, jnp.where(g1, jv, i1))
                nt2 = jnp.where(g1, t1, jnp.where(g2, v, t2))
                ni2 = jnp.where(g1, i1, jnp.where(g2, jv, i2))
                nt3 = jnp.where(g2, t2, jnp.where(g3, v, t3))
                ni3 = jnp.where(g2, i2, jnp.where(g3, jv, i3))
                return (nt0, nt1, nt2, nt3, ni0, ni1, ni2, ni3)

            res = lax.fori_loop(0, m, seg_body,
                                (neg, neg, neg, neg,
                                 iinit, iinit, iinit, iinit))
            for tt in range(TOPK):
                buf[tt, pl.ds(sg * 16, 16)] = res[4 + tt]
        pltpu.sync_copy(buf, idx_ref.at[h, :, pl.ds(g * 128, 128)])
        return carry

    lax.fori_loop(0, groups_per_w, group_body, 0)


def _attn_kernel(q_ref, k_ref, v_ref, idx_ref, r_ref, wp_ref, b_ref, o_ref,
                 *, seg, scale, hd):
    hp = pl.program_id(1)
    bq = q_ref.shape[0]
    n = k_ref.shape[0]
    m = n // seg
    seg_sub = jax.lax.broadcasted_iota(jnp.int32, (m, bq), 0)
    rbf = r_ref[...]

    @pl.when(hp == 0)
    def _():
        o_ref[...] = jnp.broadcast_to(b_ref[...], o_ref.shape)

    o_halves = []
    for half in range(2):
        sl = slice(half * hd, (half + 1) * hd)
        q = q_ref[:, sl]
        k = k_ref[:, sl]
        v = v_ref[:, sl]

        idxs = idx_ref[half]
        eqT = seg_sub == idxs[0:1, :]
        for t in range(1, TOPK):
            eqT = eqT | (seg_sub == idxs[t:t + 1, :])
        selbigT = jnp.where(eqT, BIG, 0.0)

        s = _mm(q * scale, k, ((1,), (1,)))
        s = s + _mm(selbigT, rbf, ((0,), (1,)))
        mxs = jnp.max(s, axis=1, keepdims=True)
        e = jnp.exp(s - mxs)
        p = e * (1.0 / jnp.sum(e, axis=1, keepdims=True))
        o_halves.append(_mm(p, v, ((1,), (0,))))

    o_pair = jnp.concatenate(o_halves, axis=1)
    o_ref[...] += _mm(o_pair, wp_ref[...], ((1,), (1,)))


@functools.partial(jax.jit, static_argnames=("interpret",))
def kernel(x, W_qkv, W_proj, b_proj, interpret=False):
    Bb, Nn, Cc = x.shape
    hd = Cc // H
    scale = hd ** (-0.5)
    m = min(M_LANDMARKS, Nn)
    seg = (Nn + m - 1) // m

    xf = x.reshape(Bb * Nn, Cc)
    bn = Bb * Nn
    blk = 256
    grid_a = (bn // blk,)

    qkv = pl.pallas_call(
        _qkv_kernel,
        grid=grid_a,
        in_specs=[
            pl.BlockSpec((blk, Cc), lambda i: (i, 0)),
            pl.BlockSpec((3 * Cc, Cc), lambda i: (0, 0)),
        ],
        out_specs=pl.BlockSpec((blk, 3 * Cc), lambda i: (i, 0)),
        out_shape=jax.ShapeDtypeStruct((bn, 3 * Cc), jnp.float32),
        interpret=interpret,
    )(xf, W_qkv)

    hp = H // 2

    rs_all = pl.pallas_call(
        functools.partial(_rs_kernel, seg=seg, hd=hd),
        grid=(hp,),
        in_specs=[
            pl.BlockSpec((Nn, 2 * hd), lambda h: (0, h)),
            pl.BlockSpec((Nn, 2 * hd), lambda h: (0, hp + h)),
        ],
        out_specs=pl.BlockSpec((2, m, Nn), lambda h: (h, 0, 0)),
        out_shape=jax.ShapeDtypeStruct((H, m, Nn), jnp.float32),
        interpret=interpret,
    )(qkv, qkv)

    n_workers = 32
    groups = H * (Nn // 128)
    gph = Nn // 128
    mesh = plsc.VectorSubcoreMesh(core_axis_name="c", subcore_axis_name="s",
                                  num_cores=2)
    idx_all = pl.kernel(
        functools.partial(_sc_topk, groups_per_w=groups // n_workers,
                          gph=gph, m=m),
        mesh=mesh,
        out_type=jax.ShapeDtypeStruct((H, TOPK, Nn), jnp.int32),
        scratch_types=[
            pltpu.VMEM((m, 128), jnp.float32),
            pltpu.VMEM((TOPK, 128), jnp.int32),
        ],
    )(rs_all)

    rbf = (jnp.arange(Nn, dtype=jnp.int32)[:, None] // seg
           == jnp.arange(m, dtype=jnp.int32)[None, :]).astype(BF)

    bq = 2048
    grid_b = (Nn // bq, hp)
    out = pl.pallas_call(
        functools.partial(_attn_kernel, seg=seg, scale=scale, hd=hd),
        grid=grid_b,
        in_specs=[
            pl.BlockSpec((bq, 2 * hd), lambda i, h: (i, h)),
            pl.BlockSpec((Nn, 2 * hd), lambda i, h: (0, hp + h)),
            pl.BlockSpec((Nn, 2 * hd), lambda i, h: (0, 2 * hp + h)),
            pl.BlockSpec((2, TOPK, Nn), lambda i, h: (h, 0, 0)),
            pl.BlockSpec((Nn, m), lambda i, h: (0, 0)),
            pl.BlockSpec((Cc, 2 * hd), lambda i, h: (0, h)),
            pl.BlockSpec((1, Cc), lambda i, h: (0, 0)),
        ],
        out_specs=pl.BlockSpec((bq, Cc), lambda i, h: (i, 0)),
        out_shape=jax.ShapeDtypeStruct((bn, Cc), jnp.float32),
        interpret=interpret,
    )(qkv, qkv, qkv, idx_all, rbf, W_proj, b_proj.reshape(1, Cc))

    return out.reshape(Bb, Nn, Cc)


# speedup vs baseline: 1.0066x; 1.0066x over previous
import functools

import jax
import jax.numpy as jnp
from jax import lax
from jax.experimental import pallas as pl
from jax.experimental.pallas import tpu as pltpu
from jax.experimental.pallas import tpu_sc as plsc

H = 12
M_LANDMARKS = 256
TOPK = 4
NEG = -1e30
BIG = 1024.0
BF = jnp.bfloat16


def _mm(a, b, dims):
    return jax.lax.dot_general(a.astype(BF), b.astype(BF), (dims, ((), ())),
                               preferred_element_type=jnp.float32)


def _qkv_kernel(x_ref, w_ref, o_ref):
    o_ref[...] = _mm(x_ref[...], w_ref[...], ((1,), (1,)))


def _rs_kernel(q_ref, k_ref, o_ref, *, seg, hd):
    n = k_ref.shape[0]
    m = n // seg
    for half in range(2):
        sl = slice(half * hd, (half + 1) * hd)
        k = k_ref[:, sl]
        cent = jnp.mean(k.reshape(m, seg, hd), axis=1)
        o_ref[half] = _mm(cent, q_ref[:, sl], ((1,), (1,)))


def _sc_topk(rs_ref, idx_ref, slab, buf, *, groups_per_w, gph, m):
    wid = lax.axis_index("s") * 2 + lax.axis_index("c")

    def group_body(t, carry):
        gg = wid * groups_per_w + t
        h = gg // gph
        g = gg - h * gph
        pltpu.sync_copy(rs_ref.at[h, :, pl.ds(g * 128, 128)], slab)

        neg = jnp.full((16,), NEG, jnp.float32)
        iinit = jnp.full((16,), m, jnp.int32)

        for sg in range(8):
            def seg_body(j, c, sg=sg):
                t0, t1, t2, t3, i0, i1, i2, i3 = c
                v = slab[j, pl.ds(sg * 16, 16)]
                jv = jnp.zeros((16,), jnp.int32) + j
                g0 = v > t0
                g1 = v > t1
                g2 = v > t2
                g3 = v > t3
                nt0 = jnp.where(g0, v, t0)
                ni0 = jnp.where(g0, jv, i0)
                nt1 = jnp.where(g0, t0, jnp.where(g1, v, t1))
                ni1 = jnp.where(g0, i0, jnp.where(g1, jv, i1))
                nt2 = jnp.where(g1, t1, jnp.where(g2, v, t2))
                ni2 = jnp.where(g1, i1, jnp.where(g2, jv, i2))
                nt3 = jnp.where(g2, t2, jnp.where(g3, v, t3))
                ni3 = jnp.where(g2, i2, jnp.where(g3, jv, i3))
                return (nt0, nt1, nt2, nt3, ni0, ni1, ni2, ni3)

            res = lax.fori_loop(0, m, seg_body,
                                (neg, neg, neg, neg,
                                 iinit, iinit, iinit, iinit))
            for tt in range(TOPK):
                buf[tt, pl.ds(sg * 16, 16)] = res[4 + tt]
        pltpu.sync_copy(buf, idx_ref.at[h, :, pl.ds(g * 128, 128)])
        return carry

    lax.fori_loop(0, groups_per_w, group_body, 0)


def _attn_kernel(q_ref, k_ref, v_ref, idx_ref, r_ref, wp_ref, b_ref, o_ref,
                 *, seg, scale, hd):
    hp = pl.program_id(1)
    bq = q_ref.shape[0]
    n = k_ref.shape[0]
    m = n // seg
    seg_sub = jax.lax.broadcasted_iota(jnp.int32, (m, bq), 0)
    rbf = r_ref[...]

    @pl.when(hp == 0)
    def _():
        o_ref[...] = jnp.broadcast_to(b_ref[...], o_ref.shape)

    o_halves = []
    for half in range(2):
        sl = slice(half * hd, (half + 1) * hd)
        q = q_ref[:, sl]
        k = k_ref[:, sl]
        v = v_ref[:, sl]

        idxs = idx_ref[half]
        eqT = seg_sub == idxs[0:1, :]
        for t in range(1, TOPK):
            eqT = eqT | (seg_sub == idxs[t:t + 1, :])
        selbigT = jnp.where(eqT, BIG, 0.0)

        s = _mm(q * scale, k, ((1,), (1,)))
        s = s + _mm(selbigT, rbf, ((0,), (1,)))
        mxs = jnp.max(s, axis=1, keepdims=True)
        e = jnp.exp(s - mxs)
        p = e * (1.0 / jnp.sum(e, axis=1, keepdims=True))
        o_halves.append(_mm(p, v, ((1,), (0,))))

    o_pair = jnp.concatenate(o_halves, axis=1)
    o_ref[...] += _mm(o_pair, wp_ref[...], ((1,), (1,)))


@functools.partial(jax.jit, static_argnames=("interpret",))
def kernel(x, W_qkv, W_proj, b_proj, interpret=False):
    Bb, Nn, Cc = x.shape
    hd = Cc // H
    scale = hd ** (-0.5)
    m = min(M_LANDMARKS, Nn)
    seg = (Nn + m - 1) // m

    xf = x.reshape(Bb * Nn, Cc)
    bn = Bb * Nn
    blk = 256
    grid_a = (bn // blk,)

    qkv = pl.pallas_call(
        _qkv_kernel,
        grid=grid_a,
        in_specs=[
            pl.BlockSpec((blk, Cc), lambda i: (i, 0)),
            pl.BlockSpec((3 * Cc, Cc), lambda i: (0, 0)),
        ],
        out_specs=pl.BlockSpec((blk, 3 * Cc), lambda i: (i, 0)),
        out_shape=jax.ShapeDtypeStruct((bn, 3 * Cc), jnp.float32),
        interpret=interpret,
    )(xf, W_qkv)

    hp = H // 2

    rs_all = pl.pallas_call(
        functools.partial(_rs_kernel, seg=seg, hd=hd),
        grid=(hp,),
        in_specs=[
            pl.BlockSpec((Nn, 2 * hd), lambda h: (0, h)),
            pl.BlockSpec((Nn, 2 * hd), lambda h: (0, hp + h)),
        ],
        out_specs=pl.BlockSpec((2, m, Nn), lambda h: (h, 0, 0)),
        out_shape=jax.ShapeDtypeStruct((H, m, Nn), jnp.float32),
        interpret=interpret,
    )(qkv, qkv)

    n_workers = 32
    groups = H * (Nn // 128)
    gph = Nn // 128
    mesh = plsc.VectorSubcoreMesh(core_axis_name="c", subcore_axis_name="s",
                                  num_cores=2)
    idx_all = pl.kernel(
        functools.partial(_sc_topk, groups_per_w=groups // n_workers,
                          gph=gph, m=m),
        mesh=mesh,
        out_type=jax.ShapeDtypeStruct((H, TOPK, Nn), jnp.int32),
        scratch_types=[
            pltpu.VMEM((m, 128), jnp.float32),
            pltpu.VMEM((TOPK, 128), jnp.int32),
        ],
    )(rs_all)

    rbf = (jnp.arange(Nn, dtype=jnp.int32)[:, None] // seg
           == jnp.arange(m, dtype=jnp.int32)[None, :]).astype(BF)

    bq = 2048
    grid_b = (Nn // bq, hp)
    out = pl.pallas_call(
        functools.partial(_attn_kernel, seg=seg, scale=scale, hd=hd),
        grid=grid_b,
        in_specs=[
            pl.BlockSpec((bq, 2 * hd), lambda i, h: (i, h)),
            pl.BlockSpec((Nn, 2 * hd), lambda i, h: (0, hp + h)),
            pl.BlockSpec((Nn, 2 * hd), lambda i, h: (0, 2 * hp + h)),
            pl.BlockSpec((2, TOPK, Nn), lambda i, h: (h, 0, 0)),
            pl.BlockSpec((Nn, m), lambda i, h: (0, 0)),
            pl.BlockSpec((Cc, 2 * hd), lambda i, h: (0, h)),
            pl.BlockSpec((1, Cc), lambda i, h: (0, 0)),
        ],
        out_specs=pl.BlockSpec((bq, Cc), lambda i, h: (i, 0)),
        out_shape=jax.ShapeDtypeStruct((bn, Cc), jnp.float32),
        interpret=interpret,
    )(qkv, qkv, qkv, idx_all, rbf, W_proj, b_proj.reshape(1, Cc))

    return out.reshape(Bb, Nn, Cc)
